# baseline (speedup 1.0000x reference)
import functools

import jax
import jax.numpy as jnp
from jax.experimental import pallas as pl

_SELU_SCALE = 1.0507009873554804934193349852946
_SELU_ALPHA = 1.6732632423543772848170429916717


def _body(a_ref, x_ref, w_ref, o_ref, *, d):
    t = pl.program_id(1)
    a = a_ref[0]
    x = x_ref[t]
    h = jnp.dot(a, x, preferred_element_type=jnp.float32)
    h = jnp.dot(h, w_ref[...], preferred_element_type=jnp.float32)
    h = _SELU_SCALE * jnp.where(h > 0, h, _SELU_ALPHA * (jnp.exp(h) - 1.0))
    o_ref[:, pl.ds(t * d, d)] = h


@functools.partial(jax.jit, static_argnames=("block_n",))
def _run(Feature_tensor, adj, W, block_n=256):
    n_time, n_node, d = Feature_tensor.shape
    grid = (n_node // block_n, n_time)
    out = pl.pallas_call(
        functools.partial(_body, d=d),
        grid=grid,
        in_specs=[
            pl.BlockSpec((1, block_n, n_node), lambda i, t: (t, i, 0)),
            pl.BlockSpec((n_time, n_node, d), lambda i, t: (0, 0, 0)),
            pl.BlockSpec((d, d), lambda i, t: (0, 0)),
        ],
        out_specs=pl.BlockSpec((block_n, n_time * d), lambda i, t: (i, 0)),
        out_shape=jax.ShapeDtypeStruct((n_node, n_time * d), jnp.float32),
    )(adj, Feature_tensor, W)
    return out.reshape(n_node, n_time, d)


def kernel(Feature_tensor, adj, W):
    return _run(Feature_tensor, adj, W)


# speedup vs baseline: 1.1804x; 1.1804x over previous
import functools

import jax
import jax.numpy as jnp
from jax.experimental import pallas as pl

_SELU_SCALE = 1.0507009873554804934193349852946
_SELU_ALPHA = 1.6732632423543772848170429916717


def _body(a_ref, x_ref, w_ref, o_ref):
    t = pl.program_id(1)
    a = a_ref[0].astype(jnp.bfloat16)
    x = x_ref[t].astype(jnp.bfloat16)
    h = jnp.dot(a, x, preferred_element_type=jnp.float32)
    h = jnp.dot(h.astype(jnp.bfloat16), w_ref[...].astype(jnp.bfloat16),
                preferred_element_type=jnp.float32)
    h = _SELU_SCALE * jnp.where(h > 0, h, _SELU_ALPHA * (jnp.exp(h) - 1.0))
    o_ref[:, t, :] = h


@functools.partial(jax.jit, static_argnames=("block_n",))
def _run(Feature_tensor, adj, W, block_n=256):
    n_time, n_node, d = Feature_tensor.shape
    grid = (n_node // block_n, n_time)
    return pl.pallas_call(
        _body,
        grid=grid,
        in_specs=[
            pl.BlockSpec((1, block_n, n_node), lambda i, t: (t, i, 0)),
            pl.BlockSpec((n_time, n_node, d), lambda i, t: (0, 0, 0)),
            pl.BlockSpec((d, d), lambda i, t: (0, 0)),
        ],
        out_specs=pl.BlockSpec((block_n, n_time, d), lambda i, t: (i, 0, 0)),
        out_shape=jax.ShapeDtypeStruct((n_node, n_time, d), jnp.float32),
    )(adj, Feature_tensor, W)


def kernel(Feature_tensor, adj, W):
    return _run(Feature_tensor, adj, W)


# speedup vs baseline: 1.3389x; 1.1343x over previous
import functools

import jax
import jax.numpy as jnp
from jax.experimental import pallas as pl

_SELU_SCALE = 1.0507009873554804934193349852946
_SELU_ALPHA = 1.6732632423543772848170429916717


def _body(a_ref, x_ref, w_ref, o_ref):
    a = a_ref[0].astype(jnp.bfloat16)
    x = x_ref[0].astype(jnp.bfloat16)
    h = jnp.dot(a, x, preferred_element_type=jnp.float32)
    h = jnp.dot(h.astype(jnp.bfloat16), w_ref[...].astype(jnp.bfloat16),
                preferred_element_type=jnp.float32)
    h = _SELU_SCALE * jnp.where(h > 0, h, _SELU_ALPHA * (jnp.exp(h) - 1.0))
    o_ref[...] = h


@functools.partial(jax.jit, static_argnames=("block_n",))
def _run(Feature_tensor, adj, W, block_n=512):
    n_time, n_node, d = Feature_tensor.shape
    grid = (n_time, n_node // block_n)
    out = pl.pallas_call(
        _body,
        grid=grid,
        in_specs=[
            pl.BlockSpec((1, block_n, n_node), lambda t, i: (t, i, 0)),
            pl.BlockSpec((1, n_node, d), lambda t, i: (t, 0, 0)),
            pl.BlockSpec((d, d), lambda t, i: (0, 0)),
        ],
        out_specs=pl.BlockSpec((block_n, d), lambda t, i: (i, t)),
        out_shape=jax.ShapeDtypeStruct((n_node, n_time * d), jnp.float32),
    )(adj, Feature_tensor, W)
    return out.reshape(n_node, n_time, d)


def kernel(Feature_tensor, adj, W):
    return _run(Feature_tensor, adj, W)


# speedup vs baseline: 1.4874x; 1.1109x over previous
import functools

import jax
import jax.numpy as jnp
from jax.experimental import pallas as pl

_SELU_SCALE = 1.0507009873554804934193349852946
_SELU_ALPHA = 1.6732632423543772848170429916717
_NSPLIT = 4


def _body(a0_ref, a1_ref, a2_ref, a3_ref, x_ref, w_ref, o_ref, *, block_n):
    x = x_ref[0].astype(jnp.bfloat16)
    w = w_ref[...].astype(jnp.bfloat16)
    for k, a_ref in enumerate((a0_ref, a1_ref, a2_ref, a3_ref)):
        a = a_ref[0].astype(jnp.bfloat16)
        h = jnp.dot(a, x, preferred_element_type=jnp.float32)
        h = jnp.dot(h.astype(jnp.bfloat16), w, preferred_element_type=jnp.float32)
        h = _SELU_SCALE * jnp.where(h > 0, h, _SELU_ALPHA * (jnp.exp(h) - 1.0))
        o_ref[k * block_n:(k + 1) * block_n, :] = h


@functools.partial(jax.jit, static_argnames=("block_n",))
def _run(Feature_tensor, adj, W, block_n=256):
    n_time, n_node, d = Feature_tensor.shape
    grid = (n_time, n_node // (_NSPLIT * block_n))
    adj_specs = [
        pl.BlockSpec((1, block_n, n_node),
                     functools.partial(lambda k, t, i: (t, _NSPLIT * i + k, 0), k))
        for k in range(_NSPLIT)
    ]
    out = pl.pallas_call(
        functools.partial(_body, block_n=block_n),
        grid=grid,
        in_specs=adj_specs + [
            pl.BlockSpec((1, n_node, d), lambda t, i: (t, 0, 0)),
            pl.BlockSpec((d, d), lambda t, i: (0, 0)),
        ],
        out_specs=pl.BlockSpec((_NSPLIT * block_n, d), lambda t, i: (i, t)),
        out_shape=jax.ShapeDtypeStruct((n_node, n_time * d), jnp.float32),
    )(adj, adj, adj, adj, Feature_tensor, W)
    return out.reshape(n_node, n_time, d)


def kernel(Feature_tensor, adj, W):
    return _run(Feature_tensor, adj, W)
